# baseline (speedup 1.0000x reference)
import functools

import jax
import jax.numpy as jnp
import numpy as np
from jax import lax
from jax.experimental import pallas as pl
from jax.experimental.pallas import tpu as pltpu
from jax.experimental.pallas import tpu_sc as plsc

K = 16
F = 26
B = 4096


_T_MAIN = 999936
_T_C = 1536
_T_NCHUNK = _T_MAIN // _T_C
_T_U = 8


def _sc_transpose(emb_T, tail_flat):
    n = emb_T.shape[1]
    info = plsc.get_sparse_core_info()
    nc, ns = info.num_cores, info.num_subcores
    nw = nc * ns
    iters = (_T_NCHUNK + nw - 1) // nw

    mesh = plsc.VectorSubcoreMesh(core_axis_name="c", subcore_axis_name="s")

    @functools.partial(
        pl.kernel,
        mesh=mesh,
        out_type=jax.ShapeDtypeStruct((n * K,), jnp.float32),
        scratch_types=[
            pltpu.VMEM((K, _T_C), jnp.float32),
            pltpu.VMEM((K, _T_C), jnp.float32),
            pltpu.VMEM((_T_C * K,), jnp.float32),
            pltpu.VMEM((_T_C * K,), jnp.float32),
            pltpu.SemaphoreType.DMA,
            pltpu.SemaphoreType.DMA,
            pltpu.SemaphoreType.DMA,
        ],
        compiler_params=pltpu.CompilerParams(
            use_tc_tiling_on_sc=True, needs_layout_passes=False),
    )
    def transpose_kernel(emb_hbm, tail_hbm, out_hbm, in_v0, in_v1,
                         out_v0, out_v1, sem_a, sem_b, sem_out):
        wid = lax.axis_index("s") * nc + lax.axis_index("c")
        scat_iota = lax.iota(jnp.int32, 16) * K
        idx_vecs = [scat_iota + k for k in range(K)]
        in_sems = (sem_a, sem_b)
        in_bufs = (in_v0, in_v1)
        out_bufs = (out_v0, out_v1)

        def _in_desc(ci, buf):
            return (emb_hbm.at[:, pl.ds(ci * _T_C, _T_C)], in_bufs[buf],
                    in_sems[buf])

        def _out_desc(ci, buf):
            return (out_bufs[buf],
                    out_hbm.at[pl.ds(ci * _T_C * K, _T_C * K)], sem_out)

        def in_start(ci, buf):
            pltpu.async_copy(*_in_desc(ci, buf))

        def in_wait(ci, buf):
            pltpu.make_async_copy(*_in_desc(ci, buf)).wait()

        def out_start(ci, buf):
            pltpu.async_copy(*_out_desc(ci, buf))

        def out_wait(ci, buf):
            pltpu.make_async_copy(*_out_desc(ci, buf)).wait()

        @pl.when(wid < _T_NCHUNK)
        def _():
            in_start(wid, 0)

        for t in range(iters):
            ci = wid + t * nw
            cur = t % 2

            @pl.when(ci < _T_NCHUNK)
            def _():
                in_wait(ci, cur)
                @pl.when(ci + nw < _T_NCHUNK)
                def _():
                    in_start(ci + nw, 1 - cur)

                src = in_bufs[cur]
                dst = out_bufs[cur]

                def row_body(j, _):
                    dslice = dst.at[pl.ds(j * 256, 256)]
                    vs = [src[k, pl.ds(j * 16, 16)] for k in range(K)]
                    for k in range(K):
                        plsc.store_scatter(dslice, [idx_vecs[k]], vs[k])
                    return 0

                lax.fori_loop(0, _T_C // 16, row_body, 0)

                if t > 0:
                    out_wait(ci - nw, 1 - cur)
                out_start(ci, cur)

        last_t = iters - 1
        last_ci = wid + last_t * nw
        is_last_valid = last_ci < _T_NCHUNK
        @pl.when(is_last_valid)
        def _():
            out_wait(last_ci, last_t % 2)
        @pl.when(jnp.logical_not(is_last_valid))
        def _():
            out_wait(wid + (last_t - 1) * nw, (last_t - 1) % 2)

        @pl.when(wid == 0)
        def _():
            tail_n = (n - _T_MAIN) * K
            pltpu.sync_copy(tail_hbm, out_v0.at[pl.ds(0, tail_n)])
            pltpu.sync_copy(out_v0.at[pl.ds(0, tail_n)],
                            out_hbm.at[pl.ds(_T_MAIN * K, tail_n)])

    return transpose_kernel(emb_T, tail_flat)


def _sc_gather(x_flat, emb_v, w1_flat):
    info = plsc.get_sparse_core_info()
    nc, ns = info.num_cores, info.num_subcores
    nw = nc * ns
    bf = x_flat.shape[0]
    per_w = bf // nw
    assert per_w * nw == bf and per_w % 8 == 0

    mesh = plsc.VectorSubcoreMesh(core_axis_name="c", subcore_axis_name="s")

    @functools.partial(
        pl.kernel,
        mesh=mesh,
        out_type=[
            jax.ShapeDtypeStruct((bf, K), jnp.float32),
            jax.ShapeDtypeStruct((bf,), jnp.float32),
        ],
        scratch_types=[
            pltpu.VMEM((per_w,), jnp.int32),
            pltpu.VMEM((per_w, K), jnp.float32),
            pltpu.VMEM((per_w,), jnp.float32),
            pltpu.SemaphoreType.DMA,
            pltpu.SemaphoreType.DMA,
        ],
        compiler_params=pltpu.CompilerParams(use_tc_tiling_on_sc=False),
    )
    def gather_kernel(x_hbm, emb_hbm, w1_hbm, rows_out, w1_out,
                      idx_v, rows_v, w1_v, sem_r, sem_w):
        wid = lax.axis_index("s") * nc + lax.axis_index("c")
        base = wid * per_w
        pltpu.sync_copy(x_hbm.at[pl.ds(base, per_w)], idx_v)
        cp_r = pltpu.async_copy(emb_hbm.at[idx_v], rows_v, sem_r)
        cp_w = pltpu.async_copy(w1_hbm.at[idx_v], w1_v, sem_w)
        cp_r.wait()
        cp_w.wait()
        pltpu.sync_copy(rows_v, rows_out.at[pl.ds(base, per_w)])
        pltpu.sync_copy(w1_v, w1_out.at[pl.ds(base, per_w)])

    return gather_kernel(x_flat, emb_v, w1_flat)


def _tc_body(di_ref, w1v_ref, W1_ref, b1_ref, W2_ref, b2_ref, W3_ref, b3_ref,
             Wh_ref, cvec_ref, scal_ref, out_ref):
    mm = functools.partial(
        lax.dot_general,
        dimension_numbers=(((1,), (0,)), ((), ())),
        preferred_element_type=jnp.float32,
        precision=lax.Precision.DEFAULT,
    )
    bf = jnp.bfloat16
    di = di_ref[...]
    h = jnp.maximum(mm(di.astype(bf), W1_ref[...].astype(bf)) + b1_ref[...], 0.0)
    h = jnp.maximum(mm(h.astype(bf), W2_ref[...].astype(bf)) + b2_ref[...], 0.0)
    h = jnp.maximum(mm(h.astype(bf), W3_ref[...].astype(bf)) + b3_ref[...], 0.0)
    fm2 = jnp.sum(di * di * cvec_ref[...], axis=1, keepdims=True)
    fm1 = jnp.sum(w1v_ref[...], axis=1, keepdims=True)
    wfm = scal_ref[0]
    c0 = scal_ref[1]
    logit = mm(h, Wh_ref[...]) + (fm1 + fm2) * wfm + c0
    out_ref[...] = jax.nn.sigmoid(logit)


def _tc_dense(di, w1v, W1, b1, W2, b2, W3, b3, Wh, cvec, scal):
    blk = 512
    nb = B // blk
    d_in = di.shape[1]
    d_h = W2.shape[0]
    const = lambda i: (0, 0)
    return pl.pallas_call(
        _tc_body,
        grid=(nb,),
        in_specs=[
            pl.BlockSpec((blk, d_in), lambda i: (i, 0)),
            pl.BlockSpec((blk, F), lambda i: (i, 0)),
            pl.BlockSpec((d_in, d_h), const),
            pl.BlockSpec((1, d_h), const),
            pl.BlockSpec((d_h, d_h), const),
            pl.BlockSpec((1, d_h), const),
            pl.BlockSpec((d_h, d_h), const),
            pl.BlockSpec((1, d_h), const),
            pl.BlockSpec((d_h, 1), const),
            pl.BlockSpec((1, d_in), const),
            pl.BlockSpec(memory_space=pltpu.SMEM),
        ],
        out_specs=pl.BlockSpec((blk, 1), lambda i: (i, 0)),
        out_shape=jax.ShapeDtypeStruct((B, 1), jnp.float32),
        compiler_params=pltpu.CompilerParams(
            dimension_semantics=("arbitrary",),
        ),
    )(di, w1v, W1, b1, W2, b2, W3, b3, Wh, cvec, scal)


def kernel(x, emb_v, w0, w1, W1, b1, W2, b2, W3, b3, W_out, b_out):
    x_flat = x.reshape(-1)
    w1_flat = w1.reshape(-1)
    tail_flat = emb_v[_T_MAIN:].reshape(-1)
    emb_rowmajor = _sc_transpose(emb_v.T, tail_flat).reshape(emb_v.shape)
    rows, w1_vals = _sc_gather(x_flat, emb_rowmajor, w1_flat)
    di = rows.reshape(B, F * K)
    w1v = w1_vals.reshape(B, F)

    counts = (F - 1 - np.arange(F)).astype(np.float32)
    cvec = jnp.asarray(np.repeat(counts, K)[None, :])
    Wh = W_out[:-1]
    wfm = W_out[-1, 0]
    c0 = w0 * wfm + b_out[0]
    scal = jnp.stack([wfm, c0])

    return _tc_dense(di, w1v, W1, b1.reshape(1, -1), W2, b2.reshape(1, -1),
                     W3, b3.reshape(1, -1), Wh, cvec, scal)


# speedup vs baseline: 1.0021x; 1.0021x over previous
import functools

import jax
import jax.numpy as jnp
import numpy as np
from jax import lax
from jax.experimental import pallas as pl
from jax.experimental.pallas import tpu as pltpu
from jax.experimental.pallas import tpu_sc as plsc

K = 16
F = 26
B = 4096


_T_MAIN = 999936
_T_C = 1536
_T_NCHUNK = _T_MAIN // _T_C
_T_U = 8


def _sc_transpose(emb_T, tail_flat):
    n = emb_T.shape[1]
    info = plsc.get_sparse_core_info()
    nc, ns = info.num_cores, info.num_subcores
    nw = nc * ns
    iters = (_T_NCHUNK + nw - 1) // nw

    mesh = plsc.VectorSubcoreMesh(core_axis_name="c", subcore_axis_name="s")

    @functools.partial(
        pl.kernel,
        mesh=mesh,
        out_type=jax.ShapeDtypeStruct((n * K,), jnp.float32),
        scratch_types=[
            pltpu.VMEM((K, _T_C), jnp.float32),
            pltpu.VMEM((K, _T_C), jnp.float32),
            pltpu.VMEM((_T_C * K,), jnp.float32),
            pltpu.VMEM((_T_C * K,), jnp.float32),
            pltpu.SemaphoreType.DMA,
            pltpu.SemaphoreType.DMA,
            pltpu.SemaphoreType.DMA,
        ],
        compiler_params=pltpu.CompilerParams(
            use_tc_tiling_on_sc=True, needs_layout_passes=False),
    )
    def transpose_kernel(emb_hbm, tail_hbm, out_hbm, in_v0, in_v1,
                         out_v0, out_v1, sem_a, sem_b, sem_out):
        wid = lax.axis_index("s") * nc + lax.axis_index("c")
        scat_iota = lax.iota(jnp.int32, 16) * K
        idx_vecs = [scat_iota + k for k in range(K)]
        in_sems = (sem_a, sem_b)
        in_bufs = (in_v0, in_v1)
        out_bufs = (out_v0, out_v1)

        def _in_desc(ci, buf):
            return (emb_hbm.at[:, pl.ds(ci * _T_C, _T_C)], in_bufs[buf],
                    in_sems[buf])

        def _out_desc(ci, buf):
            return (out_bufs[buf],
                    out_hbm.at[pl.ds(ci * _T_C * K, _T_C * K)], sem_out)

        def in_start(ci, buf):
            pltpu.async_copy(*_in_desc(ci, buf))

        def in_wait(ci, buf):
            pltpu.make_async_copy(*_in_desc(ci, buf)).wait()

        def out_start(ci, buf):
            pltpu.async_copy(*_out_desc(ci, buf))

        def out_wait(ci, buf):
            pltpu.make_async_copy(*_out_desc(ci, buf)).wait()

        @pl.when(wid < _T_NCHUNK)
        def _():
            in_start(wid, 0)

        for t in range(iters):
            ci = wid + t * nw
            cur = t % 2

            @pl.when(ci < _T_NCHUNK)
            def _():
                in_wait(ci, cur)
                @pl.when(ci + nw < _T_NCHUNK)
                def _():
                    in_start(ci + nw, 1 - cur)

                src = in_bufs[cur]
                dst = out_bufs[cur]

                def row_body(j, _):
                    dslice = dst.at[pl.ds(j * 256, 256)]
                    vs = [src[k, pl.ds(j * 16, 16)] for k in range(K)]
                    for k in range(K):
                        plsc.store_scatter(dslice, [idx_vecs[k]], vs[k])
                    return 0

                lax.fori_loop(0, _T_C // 16, row_body, 0)

                if t > 0:
                    out_wait(ci - nw, 1 - cur)
                out_start(ci, cur)

        last_t = iters - 1
        last_ci = wid + last_t * nw
        is_last_valid = last_ci < _T_NCHUNK
        @pl.when(is_last_valid)
        def _():
            out_wait(last_ci, last_t % 2)
        @pl.when(jnp.logical_not(is_last_valid))
        def _():
            out_wait(wid + (last_t - 1) * nw, (last_t - 1) % 2)

        @pl.when(wid == 0)
        def _():
            tail_n = (n - _T_MAIN) * K
            pltpu.sync_copy(tail_hbm, out_v0.at[pl.ds(0, tail_n)])
            pltpu.sync_copy(out_v0.at[pl.ds(0, tail_n)],
                            out_hbm.at[pl.ds(_T_MAIN * K, tail_n)])

    return transpose_kernel(emb_T, tail_flat)


def _sc_gather(x_flat, emb_v, w1_flat):
    info = plsc.get_sparse_core_info()
    nc, ns = info.num_cores, info.num_subcores
    nw = nc * ns
    bf = x_flat.shape[0]
    per_w = bf // nw
    assert per_w * nw == bf and per_w % 8 == 0

    mesh = plsc.VectorSubcoreMesh(core_axis_name="c", subcore_axis_name="s")

    @functools.partial(
        pl.kernel,
        mesh=mesh,
        out_type=[
            jax.ShapeDtypeStruct((bf, K), jnp.float32),
            jax.ShapeDtypeStruct((bf,), jnp.float32),
        ],
        scratch_types=[
            pltpu.VMEM((per_w,), jnp.int32),
            pltpu.VMEM((per_w, K), jnp.float32),
            pltpu.VMEM((per_w,), jnp.float32),
            pltpu.SemaphoreType.DMA,
            pltpu.SemaphoreType.DMA,
        ],
        compiler_params=pltpu.CompilerParams(use_tc_tiling_on_sc=False),
    )
    def gather_kernel(x_hbm, emb_hbm, w1_hbm, rows_out, w1_out,
                      idx_v, rows_v, w1_v, sem_r, sem_w):
        wid = lax.axis_index("s") * nc + lax.axis_index("c")
        base = wid * per_w
        pltpu.sync_copy(x_hbm.at[pl.ds(base, per_w)], idx_v)
        cp_r = pltpu.async_copy(emb_hbm.at[idx_v], rows_v, sem_r)
        cp_w = pltpu.async_copy(w1_hbm.at[idx_v], w1_v, sem_w)
        cp_r.wait()
        cp_w.wait()
        pltpu.sync_copy(rows_v, rows_out.at[pl.ds(base, per_w)])
        pltpu.sync_copy(w1_v, w1_out.at[pl.ds(base, per_w)])

    return gather_kernel(x_flat, emb_v, w1_flat)


def _tc_body(di_ref, w1v_ref, W1_ref, b1_ref, W2_ref, b2_ref, W3_ref, b3_ref,
             Wh_ref, cvec_ref, scal_ref, out_ref):
    mm = functools.partial(
        lax.dot_general,
        dimension_numbers=(((1,), (0,)), ((), ())),
        preferred_element_type=jnp.float32,
        precision=lax.Precision.DEFAULT,
    )
    di = di_ref[...]
    h = jnp.maximum(mm(di, W1_ref[...]) + b1_ref[...], 0.0)
    h = jnp.maximum(mm(h, W2_ref[...]) + b2_ref[...], 0.0)
    h = jnp.maximum(mm(h, W3_ref[...]) + b3_ref[...], 0.0)
    fm2 = jnp.sum(di * di * cvec_ref[...], axis=1, keepdims=True)
    fm1 = jnp.sum(w1v_ref[...], axis=1, keepdims=True)
    wfm = scal_ref[0]
    c0 = scal_ref[1]
    logit = mm(h, Wh_ref[...]) + (fm1 + fm2) * wfm + c0
    out_ref[...] = jax.nn.sigmoid(logit)


def _tc_dense(di, w1v, W1, b1, W2, b2, W3, b3, Wh, cvec, scal):
    blk = 512
    nb = B // blk
    d_in = di.shape[1]
    d_h = W2.shape[0]
    const = lambda i: (0, 0)
    return pl.pallas_call(
        _tc_body,
        grid=(nb,),
        in_specs=[
            pl.BlockSpec((blk, d_in), lambda i: (i, 0)),
            pl.BlockSpec((blk, F), lambda i: (i, 0)),
            pl.BlockSpec((d_in, d_h), const),
            pl.BlockSpec((1, d_h), const),
            pl.BlockSpec((d_h, d_h), const),
            pl.BlockSpec((1, d_h), const),
            pl.BlockSpec((d_h, d_h), const),
            pl.BlockSpec((1, d_h), const),
            pl.BlockSpec((d_h, 1), const),
            pl.BlockSpec((1, d_in), const),
            pl.BlockSpec(memory_space=pltpu.SMEM),
        ],
        out_specs=pl.BlockSpec((blk, 1), lambda i: (i, 0)),
        out_shape=jax.ShapeDtypeStruct((B, 1), jnp.float32),
        compiler_params=pltpu.CompilerParams(
            dimension_semantics=("arbitrary",),
        ),
    )(di, w1v, W1, b1, W2, b2, W3, b3, Wh, cvec, scal)


def kernel(x, emb_v, w0, w1, W1, b1, W2, b2, W3, b3, W_out, b_out):
    x_flat = x.reshape(-1)
    w1_flat = w1.reshape(-1)
    tail_flat = emb_v[_T_MAIN:].reshape(-1)
    emb_rowmajor = _sc_transpose(emb_v.T, tail_flat).reshape(emb_v.shape)
    rows, w1_vals = _sc_gather(x_flat, emb_rowmajor, w1_flat)
    di = rows.reshape(B, F * K)
    w1v = w1_vals.reshape(B, F)

    counts = (F - 1 - np.arange(F)).astype(np.float32)
    cvec = jnp.asarray(np.repeat(counts, K)[None, :])
    Wh = W_out[:-1]
    wfm = W_out[-1, 0]
    c0 = w0 * wfm + b_out[0]
    scal = jnp.stack([wfm, c0])

    return _tc_dense(di, w1v, W1, b1.reshape(1, -1), W2, b2.reshape(1, -1),
                     W3, b3.reshape(1, -1), Wh, cvec, scal)


# speedup vs baseline: 1.0150x; 1.0129x over previous
import functools

import jax
import jax.numpy as jnp
import numpy as np
from jax import lax
from jax.experimental import pallas as pl
from jax.experimental.pallas import tpu as pltpu
from jax.experimental.pallas import tpu_sc as plsc

K = 16
F = 26
B = 4096


_T_MAIN = 999936
_T_C = 1792
_T_NCHUNK = _T_MAIN // _T_C
_T_U = 8


def _sc_transpose(emb_T, tail_flat):
    n = emb_T.shape[1]
    info = plsc.get_sparse_core_info()
    nc, ns = info.num_cores, info.num_subcores
    nw = nc * ns
    iters = (_T_NCHUNK + nw - 1) // nw

    mesh = plsc.VectorSubcoreMesh(core_axis_name="c", subcore_axis_name="s")

    @functools.partial(
        pl.kernel,
        mesh=mesh,
        out_type=jax.ShapeDtypeStruct((n * K,), jnp.float32),
        scratch_types=[
            pltpu.VMEM((K, _T_C), jnp.float32),
            pltpu.VMEM((K, _T_C), jnp.float32),
            pltpu.VMEM((_T_C * K,), jnp.float32),
            pltpu.VMEM((_T_C * K,), jnp.float32),
            pltpu.SemaphoreType.DMA,
            pltpu.SemaphoreType.DMA,
            pltpu.SemaphoreType.DMA,
        ],
        compiler_params=pltpu.CompilerParams(
            use_tc_tiling_on_sc=True, needs_layout_passes=False),
    )
    def transpose_kernel(emb_hbm, tail_hbm, out_hbm, in_v0, in_v1,
                         out_v0, out_v1, sem_a, sem_b, sem_out):
        wid = lax.axis_index("s") * nc + lax.axis_index("c")
        scat_iota = lax.iota(jnp.int32, 16) * K
        idx_vecs = [scat_iota + k for k in range(K)]
        in_sems = (sem_a, sem_b)
        in_bufs = (in_v0, in_v1)
        out_bufs = (out_v0, out_v1)

        def _in_desc(ci, buf):
            return (emb_hbm.at[:, pl.ds(ci * _T_C, _T_C)], in_bufs[buf],
                    in_sems[buf])

        def _out_desc(ci, buf):
            return (out_bufs[buf],
                    out_hbm.at[pl.ds(ci * _T_C * K, _T_C * K)], sem_out)

        def in_start(ci, buf):
            pltpu.async_copy(*_in_desc(ci, buf))

        def in_wait(ci, buf):
            pltpu.make_async_copy(*_in_desc(ci, buf)).wait()

        def out_start(ci, buf):
            pltpu.async_copy(*_out_desc(ci, buf))

        def out_wait(ci, buf):
            pltpu.make_async_copy(*_out_desc(ci, buf)).wait()

        @pl.when(wid < _T_NCHUNK)
        def _():
            in_start(wid, 0)

        for t in range(iters):
            ci = wid + t * nw
            cur = t % 2

            @pl.when(ci < _T_NCHUNK)
            def _():
                in_wait(ci, cur)
                @pl.when(ci + nw < _T_NCHUNK)
                def _():
                    in_start(ci + nw, 1 - cur)

                src = in_bufs[cur]
                dst = out_bufs[cur]

                def row_body(j, _):
                    dslice = dst.at[pl.ds(j * 256, 256)]
                    vs = [src[k, pl.ds(j * 16, 16)] for k in range(K)]
                    for k in range(K):
                        plsc.store_scatter(dslice, [idx_vecs[k]], vs[k])
                    return 0

                lax.fori_loop(0, _T_C // 16, row_body, 0)

                if t > 0:
                    out_wait(ci - nw, 1 - cur)
                out_start(ci, cur)

        last_t = iters - 1
        last_ci = wid + last_t * nw
        is_last_valid = last_ci < _T_NCHUNK
        @pl.when(is_last_valid)
        def _():
            out_wait(last_ci, last_t % 2)
        @pl.when(jnp.logical_not(is_last_valid))
        def _():
            out_wait(wid + (last_t - 1) * nw, (last_t - 1) % 2)

        @pl.when(wid == 0)
        def _():
            tail_n = (n - _T_MAIN) * K
            pltpu.sync_copy(tail_hbm, out_v0.at[pl.ds(0, tail_n)])
            pltpu.sync_copy(out_v0.at[pl.ds(0, tail_n)],
                            out_hbm.at[pl.ds(_T_MAIN * K, tail_n)])

    return transpose_kernel(emb_T, tail_flat)


def _sc_gather(x_flat, emb_v, w1_flat):
    info = plsc.get_sparse_core_info()
    nc, ns = info.num_cores, info.num_subcores
    nw = nc * ns
    bf = x_flat.shape[0]
    per_w = bf // nw
    assert per_w * nw == bf and per_w % 8 == 0

    mesh = plsc.VectorSubcoreMesh(core_axis_name="c", subcore_axis_name="s")

    @functools.partial(
        pl.kernel,
        mesh=mesh,
        out_type=[
            jax.ShapeDtypeStruct((bf, K), jnp.float32),
            jax.ShapeDtypeStruct((bf,), jnp.float32),
        ],
        scratch_types=[
            pltpu.VMEM((per_w,), jnp.int32),
            pltpu.VMEM((per_w, K), jnp.float32),
            pltpu.VMEM((per_w,), jnp.float32),
            pltpu.SemaphoreType.DMA,
            pltpu.SemaphoreType.DMA,
        ],
        compiler_params=pltpu.CompilerParams(use_tc_tiling_on_sc=False),
    )
    def gather_kernel(x_hbm, emb_hbm, w1_hbm, rows_out, w1_out,
                      idx_v, rows_v, w1_v, sem_r, sem_w):
        wid = lax.axis_index("s") * nc + lax.axis_index("c")
        base = wid * per_w
        pltpu.sync_copy(x_hbm.at[pl.ds(base, per_w)], idx_v)
        cp_r = pltpu.async_copy(emb_hbm.at[idx_v], rows_v, sem_r)
        cp_w = pltpu.async_copy(w1_hbm.at[idx_v], w1_v, sem_w)
        cp_r.wait()
        cp_w.wait()
        pltpu.sync_copy(rows_v, rows_out.at[pl.ds(base, per_w)])
        pltpu.sync_copy(w1_v, w1_out.at[pl.ds(base, per_w)])

    return gather_kernel(x_flat, emb_v, w1_flat)


def _tc_body(di_ref, w1v_ref, W1_ref, b1_ref, W2_ref, b2_ref, W3_ref, b3_ref,
             Wh_ref, cvec_ref, scal_ref, out_ref):
    mm = functools.partial(
        lax.dot_general,
        dimension_numbers=(((1,), (0,)), ((), ())),
        preferred_element_type=jnp.float32,
        precision=lax.Precision.DEFAULT,
    )
    di = di_ref[...]
    h = jnp.maximum(mm(di, W1_ref[...]) + b1_ref[...], 0.0)
    h = jnp.maximum(mm(h, W2_ref[...]) + b2_ref[...], 0.0)
    h = jnp.maximum(mm(h, W3_ref[...]) + b3_ref[...], 0.0)
    fm2 = jnp.sum(di * di * cvec_ref[...], axis=1, keepdims=True)
    fm1 = jnp.sum(w1v_ref[...], axis=1, keepdims=True)
    wfm = scal_ref[0]
    c0 = scal_ref[1]
    logit = mm(h, Wh_ref[...]) + (fm1 + fm2) * wfm + c0
    out_ref[...] = jax.nn.sigmoid(logit)


def _tc_dense(di, w1v, W1, b1, W2, b2, W3, b3, Wh, cvec, scal):
    blk = 512
    nb = B // blk
    d_in = di.shape[1]
    d_h = W2.shape[0]
    const = lambda i: (0, 0)
    return pl.pallas_call(
        _tc_body,
        grid=(nb,),
        in_specs=[
            pl.BlockSpec((blk, d_in), lambda i: (i, 0)),
            pl.BlockSpec((blk, F), lambda i: (i, 0)),
            pl.BlockSpec((d_in, d_h), const),
            pl.BlockSpec((1, d_h), const),
            pl.BlockSpec((d_h, d_h), const),
            pl.BlockSpec((1, d_h), const),
            pl.BlockSpec((d_h, d_h), const),
            pl.BlockSpec((1, d_h), const),
            pl.BlockSpec((d_h, 1), const),
            pl.BlockSpec((1, d_in), const),
            pl.BlockSpec(memory_space=pltpu.SMEM),
        ],
        out_specs=pl.BlockSpec((blk, 1), lambda i: (i, 0)),
        out_shape=jax.ShapeDtypeStruct((B, 1), jnp.float32),
        compiler_params=pltpu.CompilerParams(
            dimension_semantics=("arbitrary",),
        ),
    )(di, w1v, W1, b1, W2, b2, W3, b3, Wh, cvec, scal)


def kernel(x, emb_v, w0, w1, W1, b1, W2, b2, W3, b3, W_out, b_out):
    x_flat = x.reshape(-1)
    w1_flat = w1.reshape(-1)
    tail_flat = emb_v[_T_MAIN:].reshape(-1)
    emb_rowmajor = _sc_transpose(emb_v.T, tail_flat).reshape(emb_v.shape)
    rows, w1_vals = _sc_gather(x_flat, emb_rowmajor, w1_flat)
    di = rows.reshape(B, F * K)
    w1v = w1_vals.reshape(B, F)

    counts = (F - 1 - np.arange(F)).astype(np.float32)
    cvec = jnp.asarray(np.repeat(counts, K)[None, :])
    Wh = W_out[:-1]
    wfm = W_out[-1, 0]
    c0 = w0 * wfm + b_out[0]
    scal = jnp.stack([wfm, c0])

    return _tc_dense(di, w1v, W1, b1.reshape(1, -1), W2, b2.reshape(1, -1),
                     W3, b3.reshape(1, -1), Wh, cvec, scal)


# speedup vs baseline: 1.0190x; 1.0039x over previous
import functools

import jax
import jax.numpy as jnp
import numpy as np
from jax import lax
from jax.experimental import pallas as pl
from jax.experimental.pallas import tpu as pltpu
from jax.experimental.pallas import tpu_sc as plsc

K = 16
F = 26
B = 4096


_T_MAIN = 999936
_T_C = 1792
_T_NCHUNK = _T_MAIN // _T_C


def _sc_transpose(emb_T, tail_flat):
    n = emb_T.shape[1]
    info = plsc.get_sparse_core_info()
    nc, ns = info.num_cores, info.num_subcores
    nw = nc * ns
    iters = (_T_NCHUNK + nw - 1) // nw

    mesh = plsc.VectorSubcoreMesh(core_axis_name="c", subcore_axis_name="s")

    @functools.partial(
        pl.kernel,
        mesh=mesh,
        out_type=jax.ShapeDtypeStruct((n * K,), jnp.float32),
        scratch_types=[
            pltpu.VMEM((K, _T_C), jnp.float32),
            pltpu.VMEM((K, _T_C), jnp.float32),
            pltpu.VMEM((_T_C * K,), jnp.float32),
            pltpu.VMEM((_T_C * K,), jnp.float32),
            pltpu.SemaphoreType.DMA,
            pltpu.SemaphoreType.DMA,
            pltpu.SemaphoreType.DMA,
        ],
        compiler_params=pltpu.CompilerParams(
            use_tc_tiling_on_sc=True, needs_layout_passes=False),
    )
    def transpose_kernel(emb_hbm, tail_hbm, out_hbm, in_v0, in_v1,
                         out_v0, out_v1, sem_a, sem_b, sem_out):
        wid = lax.axis_index("s") * nc + lax.axis_index("c")
        scat_iota = lax.iota(jnp.int32, 16) * K
        idx_vecs = [scat_iota + k for k in range(K)]
        in_sems = (sem_a, sem_b)
        in_bufs = (in_v0, in_v1)
        out_bufs = (out_v0, out_v1)

        def _in_desc(ci, buf):
            return (emb_hbm.at[:, pl.ds(ci * _T_C, _T_C)], in_bufs[buf],
                    in_sems[buf])

        def _out_desc(ci, buf):
            return (out_bufs[buf],
                    out_hbm.at[pl.ds(ci * _T_C * K, _T_C * K)], sem_out)

        def in_start(ci, buf):
            pltpu.async_copy(*_in_desc(ci, buf))

        def in_wait(ci, buf):
            pltpu.make_async_copy(*_in_desc(ci, buf)).wait()

        def out_start(ci, buf):
            pltpu.async_copy(*_out_desc(ci, buf))

        def out_wait(ci, buf):
            pltpu.make_async_copy(*_out_desc(ci, buf)).wait()

        @pl.when(wid < _T_NCHUNK)
        def _():
            in_start(wid, 0)

        for t in range(iters):
            ci = wid + t * nw
            cur = t % 2

            @pl.when(ci < _T_NCHUNK)
            def _():
                in_wait(ci, cur)
                @pl.when(ci + nw < _T_NCHUNK)
                def _():
                    in_start(ci + nw, 1 - cur)

                src = in_bufs[cur]
                dst = out_bufs[cur]

                def row_body(j, _):
                    dslice = dst.at[pl.ds(j * 256, 256)]
                    vs = [src[k, pl.ds(j * 16, 16)] for k in range(K)]
                    for k in range(K):
                        plsc.store_scatter(dslice, [idx_vecs[k]], vs[k])
                    return 0

                lax.fori_loop(0, _T_C // 16, row_body, 0)

                if t > 0:
                    out_wait(ci - nw, 1 - cur)
                out_start(ci, cur)

        last_t = iters - 1
        last_ci = wid + last_t * nw
        is_last_valid = last_ci < _T_NCHUNK
        @pl.when(is_last_valid)
        def _():
            out_wait(last_ci, last_t % 2)
        @pl.when(jnp.logical_not(is_last_valid))
        def _():
            out_wait(wid + (last_t - 1) * nw, (last_t - 1) % 2)

        @pl.when(wid == 0)
        def _():
            tail_n = (n - _T_MAIN) * K
            pltpu.sync_copy(tail_hbm, out_v0.at[pl.ds(0, tail_n)])
            pltpu.sync_copy(out_v0.at[pl.ds(0, tail_n)],
                            out_hbm.at[pl.ds(_T_MAIN * K, tail_n)])

    return transpose_kernel(emb_T, tail_flat)


def _sc_gather(x_flat, emb_v, w1_flat):
    info = plsc.get_sparse_core_info()
    nc, ns = info.num_cores, info.num_subcores
    nw = nc * ns
    bf = x_flat.shape[0]
    per_w = bf // nw
    assert per_w * nw == bf and per_w % 8 == 0

    mesh = plsc.VectorSubcoreMesh(core_axis_name="c", subcore_axis_name="s")

    @functools.partial(
        pl.kernel,
        mesh=mesh,
        out_type=[
            jax.ShapeDtypeStruct((bf, K), jnp.float32),
            jax.ShapeDtypeStruct((bf,), jnp.float32),
        ],
        scratch_types=[
            pltpu.VMEM((per_w,), jnp.int32),
            pltpu.VMEM((per_w, K), jnp.float32),
            pltpu.VMEM((per_w,), jnp.float32),
            pltpu.SemaphoreType.DMA,
            pltpu.SemaphoreType.DMA,
        ],
        compiler_params=pltpu.CompilerParams(use_tc_tiling_on_sc=False),
    )
    def gather_kernel(x_hbm, emb_hbm, w1_hbm, rows_out, w1_out,
                      idx_v, rows_v, w1_v, sem_r, sem_w):
        wid = lax.axis_index("s") * nc + lax.axis_index("c")
        base = wid * per_w
        pltpu.sync_copy(x_hbm.at[pl.ds(base, per_w)], idx_v)
        cp_r = pltpu.async_copy(emb_hbm.at[idx_v], rows_v, sem_r)
        cp_w = pltpu.async_copy(w1_hbm.at[idx_v], w1_v, sem_w)
        cp_r.wait()
        cp_w.wait()
        pltpu.sync_copy(rows_v, rows_out.at[pl.ds(base, per_w)])
        pltpu.sync_copy(w1_v, w1_out.at[pl.ds(base, per_w)])

    return gather_kernel(x_flat, emb_v, w1_flat)


def _tc_body(di_ref, w1v_ref, W1_ref, b1_ref, W2_ref, b2_ref, W3_ref, b3_ref,
             Wh_ref, cvec_ref, scal_ref, out_ref):
    mm = functools.partial(
        lax.dot_general,
        dimension_numbers=(((1,), (0,)), ((), ())),
        preferred_element_type=jnp.float32,
        precision=lax.Precision.DEFAULT,
    )
    di = di_ref[...]
    h = jnp.maximum(mm(di, W1_ref[...]) + b1_ref[...], 0.0)
    h = jnp.maximum(mm(h, W2_ref[...]) + b2_ref[...], 0.0)
    h = jnp.maximum(mm(h, W3_ref[...]) + b3_ref[...], 0.0)
    fm2 = jnp.sum(di * di * cvec_ref[...], axis=1, keepdims=True)
    fm1 = jnp.sum(w1v_ref[...], axis=1, keepdims=True)
    wfm = scal_ref[0]
    c0 = scal_ref[1]
    logit = mm(h, Wh_ref[...]) + (fm1 + fm2) * wfm + c0
    out_ref[...] = jax.nn.sigmoid(logit)


def _tc_dense(di, w1v, W1, b1, W2, b2, W3, b3, Wh, cvec, scal):
    blk = 512
    nb = B // blk
    d_in = di.shape[1]
    d_h = W2.shape[0]
    const = lambda i: (0, 0)
    return pl.pallas_call(
        _tc_body,
        grid=(nb,),
        in_specs=[
            pl.BlockSpec((blk, d_in), lambda i: (i, 0)),
            pl.BlockSpec((blk, F), lambda i: (i, 0)),
            pl.BlockSpec((d_in, d_h), const),
            pl.BlockSpec((1, d_h), const),
            pl.BlockSpec((d_h, d_h), const),
            pl.BlockSpec((1, d_h), const),
            pl.BlockSpec((d_h, d_h), const),
            pl.BlockSpec((1, d_h), const),
            pl.BlockSpec((d_h, 1), const),
            pl.BlockSpec((1, d_in), const),
            pl.BlockSpec(memory_space=pltpu.SMEM),
        ],
        out_specs=pl.BlockSpec((blk, 1), lambda i: (i, 0)),
        out_shape=jax.ShapeDtypeStruct((B, 1), jnp.float32),
        compiler_params=pltpu.CompilerParams(
            dimension_semantics=("arbitrary",),
        ),
    )(di, w1v, W1, b1, W2, b2, W3, b3, Wh, cvec, scal)


def kernel(x, emb_v, w0, w1, W1, b1, W2, b2, W3, b3, W_out, b_out):
    x_flat = x.reshape(-1)
    w1_flat = w1.reshape(-1)
    tail_flat = emb_v[_T_MAIN:].reshape(-1)
    emb_rowmajor = _sc_transpose(emb_v.T, tail_flat).reshape(emb_v.shape)
    rows, w1_vals = _sc_gather(x_flat, emb_rowmajor, w1_flat)
    di = rows.reshape(B, F * K)
    w1v = w1_vals.reshape(B, F)

    counts = (F - 1 - np.arange(F)).astype(np.float32)
    cvec = jnp.asarray(np.repeat(counts, K)[None, :])
    Wh = W_out[:-1]
    wfm = W_out[-1, 0]
    c0 = w0 * wfm + b_out[0]
    scal = jnp.stack([wfm, c0])

    return _tc_dense(di, w1v, W1, b1.reshape(1, -1), W2, b2.reshape(1, -1),
                     W3, b3.reshape(1, -1), Wh, cvec, scal)
